# baseline (speedup 1.0000x reference)
import jax
import jax.numpy as jnp
from jax.experimental import pallas as pl
from jax.experimental.pallas import tpu as pltpu

_BUFF = 16
_PPF = 80
_PATCH2 = 9
_C = 128
_H = 128
_W = 128
_DS = 4
_FLS_H = 512.0
_FLS_W = 512.0
_R_MIN = 0.5
_R_MAX = 30.0
_FOV_H = 130.0
_PK = _C * _PATCH2


def _body(scal_ref, ts_ref, coords_ref, tb_ref, psb_ref, sfb_ref,
          fmap_hbm, imap_hbm, f1b_hbm, ib_hbm, f2b_hbm, pat_hbm, pb_hbm,
          f1o_hbm, f2o_hbm, io_hbm, po_hbm, pso_ref, to_ref, sfo_ref,
          fvm, pooled_vm, sem, sem_new):
    li = scal_ref[0]
    fn = scal_ref[1]

    cp_f = pltpu.make_async_copy(fmap_hbm.at[0], fvm, sem_new)
    cp_f.start()
    cp_i = pltpu.make_async_copy(imap_hbm.at[0], io_hbm.at[li], sem)
    cp_i.start()
    cp_p = pltpu.make_async_copy(pat_hbm.at[0], po_hbm.at[li], sem)
    cp_p.start()

    def _issue(r, carry):
        @pl.when(r != li)
        def _():
            pltpu.make_async_copy(f1b_hbm.at[r], f1o_hbm.at[r], sem).start()
            pltpu.make_async_copy(ib_hbm.at[r], io_hbm.at[r], sem).start()
            pltpu.make_async_copy(f2b_hbm.at[r], f2o_hbm.at[r], sem).start()
            pltpu.make_async_copy(pb_hbm.at[r], po_hbm.at[r], sem).start()
        return carry

    jax.lax.fori_loop(0, _BUFF, _issue, 0)

    lanes = jax.lax.broadcasted_iota(jnp.int32, (1, _BUFF), 1)
    to_ref[...] = jnp.where(lanes == li, ts_ref[0, 0], tb_ref[...])

    rows = jax.lax.broadcasted_iota(jnp.int32, (_BUFF, 1, _PPF), 0)
    sfo_ref[...] = jnp.where(rows == li, fn, sfb_ref[...])

    xy = coords_ref[0]
    rp = xy[1:2, :] * ((_R_MAX - _R_MIN) / _FLS_H) + _R_MIN
    th = (xy[0:1, :] * (1.0 / _FLS_W) - 0.5) * (_FOV_H * jnp.pi / 180.0)
    state = jnp.concatenate([rp, th, jnp.zeros((1, _PPF), jnp.float32)],
                            axis=0)
    rows3 = jax.lax.broadcasted_iota(jnp.int32, (_BUFF, 3, _PPF), 0)
    pso_ref[...] = jnp.where(rows3 == li, state[None], psb_ref[...])

    cp_f.wait()
    cp_out = pltpu.make_async_copy(fvm, f1o_hbm.at[li], sem_new)
    cp_out.start()
    _CC = 8

    def _pool(ci, carry):
        c0 = ci * _CC
        x = fvm[pl.ds(c0, _CC)]
        a = x.reshape(_CC, _H // _DS, _DS, _W).sum(axis=2)
        b = a.reshape(_CC, _H // _DS, _W // _DS, _DS).sum(axis=3)
        pooled_vm[pl.ds(c0, _CC)] = b * (1.0 / (_DS * _DS))
        return carry

    jax.lax.fori_loop(0, _C // _CC, _pool, 0)
    cp_pool = pltpu.make_async_copy(pooled_vm, f2o_hbm.at[li], sem_new)
    cp_pool.start()

    cp_i.wait()
    cp_p.wait()

    def _drain(r, carry):
        @pl.when(r != li)
        def _():
            pltpu.make_async_copy(f1b_hbm.at[r], f1o_hbm.at[r], sem).wait()
            pltpu.make_async_copy(ib_hbm.at[r], io_hbm.at[r], sem).wait()
            pltpu.make_async_copy(f2b_hbm.at[r], f2o_hbm.at[r], sem).wait()
            pltpu.make_async_copy(pb_hbm.at[r], po_hbm.at[r], sem).wait()
        return carry

    jax.lax.fori_loop(0, _BUFF, _drain, 0)
    cp_out.wait()
    cp_pool.wait()


def kernel(fmap, imap, patches, coords, time_stamp, frame_n,
           fmap1_buf, fmap2_buf, imap_buf, patches_buf,
           patch_state_buf, time_buf, source_frame_buf):
    frame_n = jnp.asarray(frame_n, jnp.int32)
    li = frame_n % _BUFF
    scal = jnp.stack([li, frame_n])

    pflat = patches.reshape(1, _PPF, _PK)
    pbflat = patches_buf.reshape(_BUFF, _PPF, _PK)
    coords2 = coords[0].T.reshape(1, 2, _PPF)
    ts2 = time_stamp.reshape(1, 1)
    ps3 = jnp.swapaxes(patch_state_buf, 1, 2)
    tb2 = time_buf.reshape(1, _BUFF)
    sf3 = source_frame_buf.reshape(_BUFF, 1, _PPF)

    f32 = jnp.float32
    any_spec = pl.BlockSpec(memory_space=pltpu.MemorySpace.HBM)
    vmem_spec = pl.BlockSpec(memory_space=pltpu.VMEM)
    smem_spec = pl.BlockSpec(memory_space=pltpu.SMEM)

    out = pl.pallas_call(
        _body,
        in_specs=[smem_spec, vmem_spec, vmem_spec, vmem_spec, vmem_spec,
                  vmem_spec,
                  any_spec, any_spec, any_spec, any_spec, any_spec,
                  any_spec, any_spec],
        out_specs=[any_spec, any_spec, any_spec, any_spec,
                   vmem_spec, vmem_spec, vmem_spec],
        out_shape=[
            jax.ShapeDtypeStruct((_BUFF, _C, _H, _W), f32),
            jax.ShapeDtypeStruct((_BUFF, _C, _H // _DS, _W // _DS), f32),
            jax.ShapeDtypeStruct((_BUFF, _C, _H, _W), f32),
            jax.ShapeDtypeStruct((_BUFF, _PPF, _PK), f32),
            jax.ShapeDtypeStruct((_BUFF, 3, _PPF), f32),
            jax.ShapeDtypeStruct((1, _BUFF), f32),
            jax.ShapeDtypeStruct((_BUFF, 1, _PPF), jnp.int32),
        ],
        scratch_shapes=[
            pltpu.VMEM((_C, _H, _W), f32),
            pltpu.VMEM((_C, _H // _DS, _W // _DS), f32),
            pltpu.SemaphoreType.DMA,
            pltpu.SemaphoreType.DMA,
        ],
    )(scal, ts2, coords2, tb2, ps3, sf3,
      fmap, imap, fmap1_buf, imap_buf, fmap2_buf, pflat, pbflat)

    f1n, f2n, imn, pnew, psnew, tnew, sfnew = out
    return (f1n, f2n, imn,
            pnew.reshape(_BUFF, _PPF, _C, _PATCH2),
            jnp.swapaxes(psnew, 1, 2),
            tnew.reshape(_BUFF),
            sfnew.reshape(_BUFF, _PPF))


# speedup vs baseline: 19.7208x; 19.7208x over previous
import jax
import jax.numpy as jnp
from jax.experimental import pallas as pl
from jax.experimental.pallas import tpu as pltpu

_BUFF = 16
_PPF = 80
_PATCH2 = 9
_C = 128
_H = 128
_W = 128
_DS = 4
_FLS_H = 512.0
_FLS_W = 512.0
_R_MIN = 0.5
_R_MAX = 30.0
_FOV_H = 130.0
_PK = _C * _PATCH2

_CB = 32
_NC = _C // _CB
_CC = 8


def _big_body(scal_ref, fmap_vm, imap_vm, f1b_ref, ib_ref, f2b_ref,
              f1o_ref, f2o_ref, io_ref):
    c = pl.program_id(0)
    r = pl.program_id(1)
    li = scal_ref[0]

    @pl.when(r == li)
    def _():
        c0 = c * _CB
        x = fmap_vm[0, pl.ds(c0, _CB)]
        f1o_ref[0] = x
        io_ref[0] = imap_vm[0, pl.ds(c0, _CB)]

        def _pool(ci, carry):
            s0 = ci * _CC
            xs = fmap_vm[0, pl.ds(c0 + s0, _CC)]
            a = xs.reshape(_CC, _H // _DS, _DS, _W).sum(axis=2)
            b = a.reshape(_CC, _H // _DS, _W // _DS, _DS).sum(axis=3)
            f2o_ref[0, pl.ds(s0, _CC)] = b * (1.0 / (_DS * _DS))
            return carry

        jax.lax.fori_loop(0, _CB // _CC, _pool, 0)

    @pl.when(r != li)
    def _():
        f1o_ref[0] = f1b_ref[0]
        io_ref[0] = ib_ref[0]
        f2o_ref[0] = f2b_ref[0]


def _small_body(scal_ref, ts_ref, coords_ref, patches_ref, pb_ref, psb_ref,
                tb_ref, sfb_ref, po_ref, pso_ref, to_ref, sfo_ref):
    r = pl.program_id(0)
    li = scal_ref[0]
    fn = scal_ref[1]

    @pl.when(r == li)
    def _():
        po_ref[0] = patches_ref[0]
        xy = coords_ref[0]
        rp = xy[1:2, :] * ((_R_MAX - _R_MIN) / _FLS_H) + _R_MIN
        th = (xy[0:1, :] * (1.0 / _FLS_W) - 0.5) * (_FOV_H * jnp.pi / 180.0)
        pso_ref[0] = jnp.concatenate(
            [rp, th, jnp.zeros((1, _PPF), jnp.float32)], axis=0)
        sfo_ref[0] = jnp.full((1, _PPF), fn, dtype=jnp.int32)

    @pl.when(r != li)
    def _():
        po_ref[0] = pb_ref[0]
        pso_ref[0] = psb_ref[0]
        sfo_ref[0] = sfb_ref[0]

    @pl.when(r == 0)
    def _():
        lanes = jax.lax.broadcasted_iota(jnp.int32, (1, _BUFF), 1)
        to_ref[...] = jnp.where(lanes == li, ts_ref[0, 0], tb_ref[...])


def kernel(fmap, imap, patches, coords, time_stamp, frame_n,
           fmap1_buf, fmap2_buf, imap_buf, patches_buf,
           patch_state_buf, time_buf, source_frame_buf):
    frame_n = jnp.asarray(frame_n, jnp.int32)
    li = frame_n % _BUFF
    scal = jnp.stack([li, frame_n])

    f32 = jnp.float32
    vmem_whole = pl.BlockSpec(memory_space=pltpu.VMEM)
    big = pl.pallas_call(
        _big_body,
        grid_spec=pltpu.PrefetchScalarGridSpec(
            num_scalar_prefetch=1,
            grid=(_NC, _BUFF),
            in_specs=[
                vmem_whole,
                vmem_whole,
                pl.BlockSpec((1, _CB, _H, _W), lambda c, r, s: (r, c, 0, 0)),
                pl.BlockSpec((1, _CB, _H, _W), lambda c, r, s: (r, c, 0, 0)),
                pl.BlockSpec((1, _CB, _H // _DS, _W // _DS),
                             lambda c, r, s: (r, c, 0, 0)),
            ],
            out_specs=[
                pl.BlockSpec((1, _CB, _H, _W), lambda c, r, s: (r, c, 0, 0)),
                pl.BlockSpec((1, _CB, _H // _DS, _W // _DS),
                             lambda c, r, s: (r, c, 0, 0)),
                pl.BlockSpec((1, _CB, _H, _W), lambda c, r, s: (r, c, 0, 0)),
            ],
        ),
        out_shape=[
            jax.ShapeDtypeStruct((_BUFF, _C, _H, _W), f32),
            jax.ShapeDtypeStruct((_BUFF, _C, _H // _DS, _W // _DS), f32),
            jax.ShapeDtypeStruct((_BUFF, _C, _H, _W), f32),
        ],
    )
    fmap1_new, fmap2_new, imap_new = big(scal, fmap, imap,
                                         fmap1_buf, imap_buf, fmap2_buf)

    pflat = patches.reshape(1, _PPF, _PK)
    pbflat = patches_buf.reshape(_BUFF, _PPF, _PK)
    coords2 = coords[0].T.reshape(1, 2, _PPF)
    ts2 = time_stamp.reshape(1, 1)
    ps3 = jnp.swapaxes(patch_state_buf, 1, 2)
    tb2 = time_buf.reshape(1, _BUFF)
    sf3 = source_frame_buf.reshape(_BUFF, 1, _PPF)

    small = pl.pallas_call(
        _small_body,
        grid_spec=pltpu.PrefetchScalarGridSpec(
            num_scalar_prefetch=1,
            grid=(_BUFF,),
            in_specs=[
                pl.BlockSpec((1, 1), lambda r, s: (0, 0)),
                pl.BlockSpec((1, 2, _PPF), lambda r, s: (0, 0, 0)),
                pl.BlockSpec((1, _PPF, _PK), lambda r, s: (0, 0, 0)),
                pl.BlockSpec((1, _PPF, _PK), lambda r, s: (r, 0, 0)),
                pl.BlockSpec((1, 3, _PPF), lambda r, s: (r, 0, 0)),
                pl.BlockSpec((1, _BUFF), lambda r, s: (0, 0)),
                pl.BlockSpec((1, 1, _PPF), lambda r, s: (r, 0, 0)),
            ],
            out_specs=[
                pl.BlockSpec((1, _PPF, _PK), lambda r, s: (r, 0, 0)),
                pl.BlockSpec((1, 3, _PPF), lambda r, s: (r, 0, 0)),
                pl.BlockSpec((1, _BUFF), lambda r, s: (0, 0)),
                pl.BlockSpec((1, 1, _PPF), lambda r, s: (r, 0, 0)),
            ],
        ),
        out_shape=[
            jax.ShapeDtypeStruct((_BUFF, _PPF, _PK), f32),
            jax.ShapeDtypeStruct((_BUFF, 3, _PPF), f32),
            jax.ShapeDtypeStruct((1, _BUFF), f32),
            jax.ShapeDtypeStruct((_BUFF, 1, _PPF), jnp.int32),
        ],
    )
    pnew, psnew, tnew, sfnew = small(scal, ts2, coords2, pflat, pbflat,
                                     ps3, tb2, sf3)

    return (fmap1_new, fmap2_new, imap_new,
            pnew.reshape(_BUFF, _PPF, _C, _PATCH2),
            jnp.swapaxes(psnew, 1, 2),
            tnew.reshape(_BUFF),
            sfnew.reshape(_BUFF, _PPF))
